# baseline (speedup 1.0000x reference)
import functools

import jax
import jax.numpy as jnp
from jax import lax
from jax.experimental import pallas as pl
from jax.experimental.pallas import tpu as pltpu
from jax.experimental.pallas import tpu_sc as plsc

B, L, H = 1024, 200, 128
LP1 = L + 1
NC, NS = 2, 16
NW = NC * NS
NB = B // NW
NSLOT = 4
EPS = 1e-12
BB = 8


@functools.partial(
    pl.kernel,
    out_type=jax.ShapeDtypeStruct((B * L, H), jnp.float32),
    mesh=plsc.VectorSubcoreMesh(core_axis_name="c", subcore_axis_name="s"),
    compiler_params=pltpu.CompilerParams(
        use_tc_tiling_on_sc=False, needs_layout_passes=False),
    scratch_types=(
        [pltpu.VMEM((NB, L), jnp.int32)]
        + [pltpu.VMEM((L, H), jnp.float32) for _ in range(NSLOT)]
        + [pltpu.SemaphoreType.DMA for _ in range(2 * NSLOT)]
    ),
)
def _gather_kernel(ids_hbm, wt_hbm, out_hbm, idx_v, *bufs_and_sems):
    rows = bufs_and_sems[:NSLOT]
    gsem = bufs_and_sems[NSLOT:2 * NSLOT]
    osem = bufs_and_sems[2 * NSLOT:]
    wid = lax.axis_index("s") * NC + lax.axis_index("c")
    base = wid * NB

    pltpu.sync_copy(ids_hbm.at[pl.ds(base, NB)], idx_v)

    def gather(s, i, make_only=False):
        mk = pltpu.make_async_copy if make_only else pltpu.async_copy
        c0 = mk(wt_hbm.at[idx_v.at[i, pl.ds(0, 104)]],
                rows[s].at[pl.ds(0, 104)], gsem[s])
        c1 = mk(wt_hbm.at[idx_v.at[i, pl.ds(104, 96)]],
                rows[s].at[pl.ds(104, 96)], gsem[s])
        return c0, c1

    def wait_gather(s, i):
        for c in gather(s, i, make_only=True):
            c.wait()

    def out_copy(s, i, make_only=False):
        mk = pltpu.make_async_copy if make_only else pltpu.async_copy
        return mk(rows[s], out_hbm.at[pl.ds((base + i) * L, L)], osem[s])

    gather(0, 0)
    gather(1, 1)

    def body(j, carry):
        for u in range(NSLOT):
            k = NSLOT * j + u
            s = u
            s2 = (u + 2) % NSLOT
            wait_gather(s, k)
            out_copy(s, k)

            @pl.when(jnp.logical_and(k + 2 < NB, k >= 2))
            def _():
                out_copy(s2, k - 2, make_only=True).wait()
                gather(s2, k + 2)

            @pl.when(jnp.logical_and(k + 2 < NB, k < 2))
            def _():
                gather(s2, k + 2)
        return carry

    lax.fori_loop(0, NB // NSLOT, body, 0, unroll=False)
    for t in range(NB - 4, NB):
        out_copy(t % NSLOT, t, make_only=True).wait()


def _ln_block(e, g, b):
    mean = jnp.mean(e, axis=-1, keepdims=True)
    var = jnp.mean(e * e, axis=-1, keepdims=True) - mean * mean
    inv = lax.rsqrt(var + EPS)
    return (e - mean) * inv * g + b


def _tc_ln_body(words_ref, vecs_ref, pos_ref, g_ref, b_ref, out_ref):
    g = g_ref[...]
    b = b_ref[...]
    pw = pos_ref[pl.ds(0, L), :]
    for bb in range(BB):
        e = words_ref[pl.ds(bb * L, L), :] + pw
        out_ref[bb, 0:L, :] = _ln_block(e, g, b)
    ev = vecs_ref[...] + pos_ref[L, :][None]
    out_ref[:, L, :] = _ln_block(ev, g, b)


_tc_ln = pl.pallas_call(
    _tc_ln_body,
    grid=(B // BB,),
    in_specs=[
        pl.BlockSpec((BB * L, H), lambda i: (i, 0)),
        pl.BlockSpec((BB, H), lambda i: (i, 0)),
        pl.BlockSpec((LP1, H), lambda i: (0, 0)),
        pl.BlockSpec((1, H), lambda i: (0, 0)),
        pl.BlockSpec((1, H), lambda i: (0, 0)),
    ],
    out_specs=pl.BlockSpec((BB, LP1, H), lambda i: (i, 0, 0)),
    out_shape=jax.ShapeDtypeStruct((B, LP1, H), jnp.float32),
)


def kernel(input_ids, vectors, word_table, pos_table, gamma, beta):
    words = _gather_kernel(input_ids.astype(jnp.int32), word_table)
    pos_sl = lax.slice(pos_table, (1, 0), (LP1 + 1, H))
    return _tc_ln(words, vectors, pos_sl,
                  gamma.reshape(1, H), beta.reshape(1, H))


# speedup vs baseline: 1.1003x; 1.1003x over previous
import functools

import jax
import jax.numpy as jnp
from jax import lax
from jax.experimental import pallas as pl
from jax.experimental.pallas import tpu as pltpu
from jax.experimental.pallas import tpu_sc as plsc

B, L, H = 1024, 200, 128
LP1 = L + 1
LPAD = 208
NC, NS = 2, 16
NW = NC * NS
NB = B // NW
NL = H // 16
EPS = 1e-12
RU = 4
RSQRT_MAGIC = 0x5F3759DF


def _rsqrt16(a):
    i = plsc.bitcast(a, jnp.int32)
    i = jnp.full((16,), RSQRT_MAGIC, dtype=jnp.int32) - lax.shift_right_logical(i, 1)
    y = plsc.bitcast(i, jnp.float32)
    half_a = a * 0.5
    for _ in range(3):
        y = y * (1.5 - half_a * y * y)
    return y


@functools.partial(
    pl.kernel,
    out_type=jax.ShapeDtypeStruct((B, LP1, H), jnp.float32),
    mesh=plsc.VectorSubcoreMesh(core_axis_name="c", subcore_axis_name="s"),
    compiler_params=pltpu.CompilerParams(
        use_tc_tiling_on_sc=True, needs_layout_passes=False),
    scratch_types=[
        pltpu.VMEM((NB * L,), jnp.int32),
        pltpu.VMEM((NB, H), jnp.float32),
        pltpu.VMEM((2, LPAD, H), jnp.float32),
        pltpu.VMEM((LPAD, H), jnp.float32),
        pltpu.VMEM((H,), jnp.float32),
        pltpu.VMEM((H,), jnp.float32),
        pltpu.SemaphoreType.DMA,
        pltpu.SemaphoreType.DMA,
        pltpu.SemaphoreType.DMA,
        pltpu.SemaphoreType.DMA,
    ],
)
def _emb_ln_kernel(ids_hbm, vec_hbm, wt_hbm, pt_hbm, g_hbm, b_hbm,
                   out_hbm, idx_v, vecs_v, rows_v, pos_v, g_v, b_v,
                   gsem0, gsem1, osem0, osem1):
    wid = lax.axis_index("s") * NC + lax.axis_index("c")
    base = wid * NB
    gsem = (gsem0, gsem1)
    osem = (osem0, osem1)

    pltpu.sync_copy(ids_hbm.at[pl.ds(pl.multiple_of(base * L, 8), NB * L)], idx_v)
    pltpu.sync_copy(vec_hbm.at[pl.ds(base, NB)], vecs_v)
    pltpu.sync_copy(pt_hbm.at[pl.ds(0, LPAD)], pos_v)
    pltpu.sync_copy(g_hbm, g_v)
    pltpu.sync_copy(b_hbm, b_v)

    gamma = [g_v[pl.ds(c * 16, 16)] for c in range(NL)]
    beta = [b_v[pl.ds(c * 16, 16)] for c in range(NL)]

    def gather_copies(p, i, make_only=False):
        mk = pltpu.make_async_copy if make_only else pltpu.async_copy
        c0 = mk(wt_hbm.at[idx_v.at[pl.ds(pl.multiple_of(i * L, 8), 104)]],
                rows_v.at[p, pl.ds(0, 104)], gsem[p])
        c1 = mk(wt_hbm.at[idx_v.at[pl.ds(pl.multiple_of(i * L + 104, 8), 96)]],
                rows_v.at[p, pl.ds(104, 96)], gsem[p])
        return c0, c1

    def wait_gather(p, i):
        for c in gather_copies(p, i, make_only=True):
            c.wait()

    def ln8(x):
        s = ((x[0] + x[1]) + (x[2] + x[3])) + ((x[4] + x[5]) + (x[6] + x[7]))
        sq = [xc * xc for xc in x]
        q = ((sq[0] + sq[1]) + (sq[2] + sq[3])) + ((sq[4] + sq[5]) + (sq[6] + sq[7]))
        tot = jnp.full((16,), jnp.sum(s))
        tot2 = jnp.full((16,), jnp.sum(q))
        mean = tot * (1.0 / H)
        var = tot2 * (1.0 / H) - mean * mean
        inv = _rsqrt16(var + EPS)
        return [(x[c] - mean) * inv * gamma[c] + beta[c] for c in range(NL)]

    def compute(p, i):
        def row_body(j, c2):
            for u in range(RU):
                l = j * RU + u
                x = [rows_v[p, l, pl.ds(c * 16, 16)] + pos_v[l + 1, pl.ds(c * 16, 16)]
                     for c in range(NL)]
                o = ln8(x)
                for c in range(NL):
                    rows_v[p, l, pl.ds(c * 16, 16)] = o[c]
            return c2

        lax.fori_loop(0, L // RU, row_body, 0, unroll=False)
        x = [vecs_v[i, pl.ds(c * 16, 16)] + pos_v[L + 1, pl.ds(c * 16, 16)]
             for c in range(NL)]
        o = ln8(x)
        for c in range(NL):
            rows_v[p, L, pl.ds(c * 16, 16)] = o[c]

    gather_copies(0, 0)

    def pair_body(j, carry):
        i0 = 2 * j
        i1 = i0 + 1

        @pl.when(j > 0)
        def _():
            pltpu.make_async_copy(rows_v.at[1, pl.ds(0, LP1)], out_hbm.at[base + i0 - 1],
                                  osem[1]).wait()

        gather_copies(1, i1)
        wait_gather(0, i0)
        compute(0, i0)
        out0 = pltpu.async_copy(rows_v.at[0, pl.ds(0, LP1)], out_hbm.at[base + i0], osem[0])
        wait_gather(1, i1)
        out0.wait()

        @pl.when(j < NB // 2 - 1)
        def _():
            gather_copies(0, i0 + 2)
        compute(1, i1)
        pltpu.async_copy(rows_v.at[1, pl.ds(0, LP1)], out_hbm.at[base + i1], osem[1])
        return carry

    lax.fori_loop(0, NB // 2, pair_body, 0, unroll=False)
    pltpu.make_async_copy(rows_v.at[1, pl.ds(0, LP1)], out_hbm.at[base + NB - 1],
                          osem[1]).wait()


def kernel(input_ids, vectors, word_table, pos_table, gamma, beta):
    return _emb_ln_kernel(input_ids.astype(jnp.int32).reshape(B * L), vectors,
                          word_table, pos_table, gamma, beta)


# speedup vs baseline: 1.1410x; 1.0370x over previous
import functools

import jax
import jax.numpy as jnp
from jax import lax
from jax.experimental import pallas as pl
from jax.experimental.pallas import tpu as pltpu
from jax.experimental.pallas import tpu_sc as plsc

B, L, H = 1024, 200, 128
LP1 = L + 1
LPAD = 208
NC, NS = 2, 16
NW = NC * NS
NB = B // NW
NL = H // 16
EPS = 1e-12
RU = 5
RSQRT_MAGIC = 0x5F3759DF


def _rsqrt16(a):
    i = plsc.bitcast(a, jnp.int32)
    i = jnp.full((16,), RSQRT_MAGIC, dtype=jnp.int32) - lax.shift_right_logical(i, 1)
    y = plsc.bitcast(i, jnp.float32)
    half_a = a * 0.5
    for _ in range(2):
        y = y * (1.5 - half_a * y * y)
    return y


@functools.partial(
    pl.kernel,
    out_type=jax.ShapeDtypeStruct((B, LP1, H), jnp.float32),
    mesh=plsc.VectorSubcoreMesh(core_axis_name="c", subcore_axis_name="s"),
    compiler_params=pltpu.CompilerParams(
        use_tc_tiling_on_sc=True, needs_layout_passes=False),
    scratch_types=[
        pltpu.VMEM((NB * L,), jnp.int32),
        pltpu.VMEM((NB, H), jnp.float32),
        pltpu.VMEM((2, LPAD, H), jnp.float32),
        pltpu.VMEM((LPAD, H), jnp.float32),
        pltpu.VMEM((H,), jnp.float32),
        pltpu.VMEM((H,), jnp.float32),
        pltpu.SemaphoreType.DMA,
        pltpu.SemaphoreType.DMA,
        pltpu.SemaphoreType.DMA,
        pltpu.SemaphoreType.DMA,
    ],
)
def _emb_ln_kernel(ids_hbm, vec_hbm, wt_hbm, pt_hbm, g_hbm, b_hbm,
                   out_hbm, idx_v, vecs_v, rows_v, pos_v, g_v, b_v,
                   gsem0, gsem1, osem0, osem1):
    wid = lax.axis_index("s") * NC + lax.axis_index("c")
    base = wid * NB
    gsem = (gsem0, gsem1)
    osem = (osem0, osem1)

    pltpu.sync_copy(ids_hbm.at[pl.ds(pl.multiple_of(base * L, 8), NB * L)], idx_v)
    pltpu.sync_copy(vec_hbm.at[pl.ds(base, NB)], vecs_v)
    pltpu.sync_copy(pt_hbm.at[pl.ds(0, LPAD)], pos_v)
    pltpu.sync_copy(g_hbm, g_v)
    pltpu.sync_copy(b_hbm, b_v)

    gamma = [g_v[pl.ds(c * 16, 16)] for c in range(NL)]
    beta = [b_v[pl.ds(c * 16, 16)] for c in range(NL)]

    def gather_copies(p, i, make_only=False):
        mk = pltpu.make_async_copy if make_only else pltpu.async_copy
        c0 = mk(wt_hbm.at[idx_v.at[pl.ds(pl.multiple_of(i * L, 8), 104)]],
                rows_v.at[p, pl.ds(0, 104)], gsem[p])
        c1 = mk(wt_hbm.at[idx_v.at[pl.ds(pl.multiple_of(i * L + 104, 8), 96)]],
                rows_v.at[p, pl.ds(104, 96)], gsem[p])
        return c0, c1

    def wait_gather(p, i):
        for c in gather_copies(p, i, make_only=True):
            c.wait()

    def ln8(x):
        s = ((x[0] + x[1]) + (x[2] + x[3])) + ((x[4] + x[5]) + (x[6] + x[7]))
        sq = [xc * xc for xc in x]
        q = ((sq[0] + sq[1]) + (sq[2] + sq[3])) + ((sq[4] + sq[5]) + (sq[6] + sq[7]))
        tot = jnp.full((16,), jnp.sum(s))
        tot2 = jnp.full((16,), jnp.sum(q))
        mean = tot * (1.0 / H)
        var = tot2 * (1.0 / H) - mean * mean
        inv = _rsqrt16(var + EPS)
        return [(x[c] - mean) * inv * gamma[c] + beta[c] for c in range(NL)]

    def compute(p, i):
        def row_body(j, c2):
            for u in range(RU):
                l = j * RU + u
                x = [rows_v[p, l, pl.ds(c * 16, 16)] + pos_v[l + 1, pl.ds(c * 16, 16)]
                     for c in range(NL)]
                o = ln8(x)
                for c in range(NL):
                    rows_v[p, l, pl.ds(c * 16, 16)] = o[c]
            return c2

        lax.fori_loop(0, L // RU, row_body, 0, unroll=False)
        x = [vecs_v[i, pl.ds(c * 16, 16)] + pos_v[L + 1, pl.ds(c * 16, 16)]
             for c in range(NL)]
        o = ln8(x)
        for c in range(NL):
            rows_v[p, L, pl.ds(c * 16, 16)] = o[c]

    gather_copies(0, 0)

    def pair_body(j, carry):
        i0 = 2 * j
        i1 = i0 + 1

        @pl.when(j > 0)
        def _():
            pltpu.make_async_copy(rows_v.at[1, pl.ds(0, LP1)], out_hbm.at[base + i0 - 1],
                                  osem[1]).wait()

        gather_copies(1, i1)
        wait_gather(0, i0)
        compute(0, i0)
        out0 = pltpu.async_copy(rows_v.at[0, pl.ds(0, LP1)], out_hbm.at[base + i0], osem[0])
        wait_gather(1, i1)
        out0.wait()

        @pl.when(j < NB // 2 - 1)
        def _():
            gather_copies(0, i0 + 2)
        compute(1, i1)
        pltpu.async_copy(rows_v.at[1, pl.ds(0, LP1)], out_hbm.at[base + i1], osem[1])
        return carry

    lax.fori_loop(0, NB // 2, pair_body, 0, unroll=False)
    pltpu.make_async_copy(rows_v.at[1, pl.ds(0, LP1)], out_hbm.at[base + NB - 1],
                          osem[1]).wait()


def kernel(input_ids, vectors, word_table, pos_table, gamma, beta):
    return _emb_ln_kernel(input_ids.astype(jnp.int32).reshape(B * L), vectors,
                          word_table, pos_table, gamma, beta)
